# baseline (speedup 1.0000x reference)
import jax
import jax.numpy as jnp
from jax import lax
from jax.experimental import pallas as pl
from jax.experimental.pallas import tpu as pltpu
from jax.experimental.pallas import tpu_sc as plsc

B, L, D = 1024, 200, 64
_MESH = plsc.VectorSubcoreMesh(core_axis_name="c", subcore_axis_name="s")


def _body(ids_hbm, char_hbm, out_hbm, idx_v, rows_v, sem):
  wid = lax.axis_index("s") * 2 + lax.axis_index("c")
  pltpu.sync_copy(ids_hbm.at[pl.ds(wid * 8, 8)], idx_v)
  pltpu.make_async_copy(char_hbm.at[idx_v.at[0]], rows_v, sem).start()
  pltpu.make_async_copy(char_hbm.at[idx_v.at[0]], rows_v, sem).wait()


@jax.jit
def _run(ids3, char2):
  return pl.kernel(
      _body,
      out_type=jax.ShapeDtypeStruct((B * 100, 128), jnp.float32),
      mesh=_MESH,
      compiler_params=pltpu.CompilerParams(
          needs_layout_passes=False, use_tc_tiling_on_sc=False),
      scratch_types=[
          pltpu.VMEM((8, 8), jnp.int32),
          pltpu.VMEM((8, 128), jnp.float32),
          pltpu.SemaphoreType.DMA,
      ],
  )(ids3, char2)


def kernel(input_ids, speaker_ids, character_embeddings, speaker_embeddings,
           ln_gamma, ln_beta):
  ids3 = jnp.right_shift(input_ids, 1).reshape(25 * B, 8)
  char2 = character_embeddings.reshape(500000, 128)
  out = _run(ids3, char2)
  return out.reshape(B, L, D)


# speedup vs baseline: 1.4288x; 1.4288x over previous
import jax
import jax.numpy as jnp
from jax import lax
from jax.experimental import pallas as pl
from jax.experimental.pallas import tpu as pltpu
from jax.experimental.pallas import tpu_sc as plsc

NUM_CORES = 2
NUM_SUBCORES = 16
LANES = 16
NW = NUM_CORES * NUM_SUBCORES

B = 1024
L = 200
D = 64
EPS = 1e-5

BPW = B // NW
NBUF = 4
KD = D // LANES

_MESH = plsc.VectorSubcoreMesh(core_axis_name="c", subcore_axis_name="s")


def _rsqrt16(x):
  i = lax.bitcast_convert_type(x, jnp.int32)
  i = jnp.int32(0x5F3759DF) - lax.shift_right_logical(i, 1)
  y = lax.bitcast_convert_type(i, jnp.float32)
  half_x = x * 0.5
  for _ in range(3):
    y = y * (1.5 - half_x * y * y)
  return y


def _body(ids_hbm, spk_ids_hbm, char_hbm, spk_emb_hbm, gamma_hbm, beta_hbm,
          out_hbm,
          all_idx, bufs, spk_rows, spk_ids_v, gamma_v, beta_v,
          gsems, osems, ssem):
  wid = lax.axis_index("s") * NUM_CORES + lax.axis_index("c")
  b0 = wid * BPW
  t0 = b0 * L

  pltpu.sync_copy(ids_hbm.at[pl.ds(t0, BPW * L)], all_idx)
  pltpu.sync_copy(spk_ids_hbm.at[pl.ds(b0, BPW)], spk_ids_v)
  pltpu.sync_copy(gamma_hbm, gamma_v)
  pltpu.sync_copy(beta_hbm, beta_v)

  for blk in range(BPW // LANES):
    sv = spk_ids_v[pl.ds(blk * LANES, LANES)]
    for u in range(LANES):
      pltpu.make_async_copy(
          spk_emb_hbm.at[sv[u]], spk_rows.at[blk * LANES + u], ssem).start()
  for r in range(BPW):
    pltpu.make_async_copy(spk_emb_hbm.at[0], spk_rows.at[r], ssem).wait()

  g = [gamma_v[pl.ds(k * LANES, LANES)] for k in range(KD)]
  bt = [beta_v[pl.ds(k * LANES, LANES)] for k in range(KD)]

  def fire(b_local, x):
    rows = bufs.at[x]
    base = b_local * L

    def m_step(m, _):
      vec = all_idx[pl.ds(base + m * LANES, LANES)]
      for u in range(LANES):
        pltpu.make_async_copy(
            char_hbm.at[vec[u]], rows.at[m * LANES + u], gsems[x]).start()
      return 0

    lax.fori_loop(0, (L // LANES), m_step, 0)
    vec = all_idx[pl.ds(base + L - LANES, LANES)]
    for u in range(LANES - (L - (L // LANES) * LANES), LANES):
      t = L - LANES + u
      pltpu.make_async_copy(char_hbm.at[vec[u]], rows.at[t], gsems[x]).start()

  def drain(x):
    pltpu.make_async_copy(char_hbm.at[pl.ds(0, L)], bufs.at[x], gsems[x]).wait()

  def compute(b_local, x):
    rows = bufs.at[x]
    spk = [spk_rows[b_local, pl.ds(k * LANES, LANES)] for k in range(KD)]

    def token(jj, u):
      t = jj * 8 + u
      v = [rows[t, pl.ds(k * LANES, LANES)] + spk[k] for k in range(KD)]
      s = (v[0] + v[1]) + (v[2] + v[3])
      q = (v[0] * v[0] + v[1] * v[1]) + (v[2] * v[2] + v[3] * v[3])
      mean = jnp.sum(s) * (1.0 / D)
      var = jnp.sum(q) * (1.0 / D) - mean * mean
      rstd = _rsqrt16(jnp.broadcast_to(var + EPS, (LANES,)))
      for k in range(KD):
        rows[t, pl.ds(k * LANES, LANES)] = (v[k] - mean) * (rstd * g[k]) + bt[k]

    def block(jj, _):
      for u in range(8):
        token(jj, u)
      return 0

    lax.fori_loop(0, L // 8, block, 0)

  def out_fire(b_local, x):
    pltpu.make_async_copy(
        bufs.at[x], out_hbm.at[pl.ds(t0 + b_local * L, L)], osems[x]).start()

  def out_wait(x):
    pltpu.make_async_copy(
        bufs.at[x], out_hbm.at[pl.ds(t0, L)], osems[x]).wait()

  for x in range(NBUF):
    fire(x, x)

  def step(i, _):
    q = NBUF * i
    for x in range(NBUF):
      drain(x)
      compute(q + x, x)
      out_fire(q + x, x)

    @pl.when(i < BPW // NBUF - 1)
    def _():
      for x in range(NBUF):
        out_wait(x)
        fire(q + NBUF + x, x)
    return 0

  lax.fori_loop(0, BPW // NBUF, step, 0)
  for x in range(NBUF):
    out_wait(x)


@jax.jit
def _run(ids_flat, speaker_ids, character_embeddings, speaker_embeddings,
         ln_gamma, ln_beta):
  return pl.kernel(
      _body,
      out_type=jax.ShapeDtypeStruct((B * L, D), jnp.float32),
      mesh=_MESH,
      compiler_params=pltpu.CompilerParams(
          needs_layout_passes=False, use_tc_tiling_on_sc=True),
      scratch_types=[
          pltpu.VMEM((BPW * L,), jnp.int32),
          pltpu.VMEM((NBUF, L, D), jnp.float32),
          pltpu.VMEM((BPW, D), jnp.float32),
          pltpu.VMEM((BPW,), jnp.int32),
          pltpu.VMEM((D,), jnp.float32),
          pltpu.VMEM((D,), jnp.float32),
          [pltpu.SemaphoreType.DMA] * NBUF,
          [pltpu.SemaphoreType.DMA] * NBUF,
          pltpu.SemaphoreType.DMA,
      ],
  )(ids_flat, speaker_ids, character_embeddings, speaker_embeddings,
    ln_gamma, ln_beta)


def kernel(input_ids, speaker_ids, character_embeddings, speaker_embeddings,
           ln_gamma, ln_beta):
  out = _run(input_ids.reshape(-1), speaker_ids, character_embeddings,
             speaker_embeddings, ln_gamma, ln_beta)
  return out.reshape(B, L, D)
